# baseline (speedup 1.0000x reference)
import dataclasses
import functools

import jax
import jax.numpy as jnp
from jax import lax
from jax.experimental import pallas as pl
from jax.experimental.pallas import tpu as pltpu
from jax.experimental.pallas import tpu_sc as plsc

_N_SEG = 10000
_NNZ = 320000
_D = 128
_EPS = 1e-16

_NC = 2
_NS = 16
_NW = _NC * _NS
_ROWS_PER_W = _NNZ // _NW
_CHUNK = 80
_NCHUNK = _ROWS_PER_W // _CHUNK
_SEG_PER_TILE = 624
_SEG_REM = _N_SEG - _NS * _SEG_PER_TILE
_ZROWS = 16


def _sc_phase1(vals_hbm, seg_hbm, part_hbm, cnt_hbm,
               table_sp, vbuf0, vbuf1, iball, hist, ztab,
               sem0, sem1, sems0, sems1):
    c = lax.axis_index("c")
    s = lax.axis_index("s")
    w = c * _NS + s

    zero16 = jnp.zeros((16,), jnp.float32)
    ones16 = zero16 + 1.0

    iload = pltpu.async_copy(seg_hbm.at[w], iball, sem0)

    @pl.loop(0, _ZROWS)
    def _(i):
        @pl.loop(0, _D // 16)
        def _(j):
            ztab[i, pl.ds(j * 16, 16)] = zero16

    @pl.loop(0, _N_SEG // 16)
    def _(i):
        hist[pl.ds(i * 16, 16)] = zero16

    @pl.loop(0, _SEG_PER_TILE // _ZROWS)
    def _(t):
        pltpu.async_copy(ztab,
                         table_sp.at[pl.ds(s * _SEG_PER_TILE + t * _ZROWS,
                                           _ZROWS)], sem1)

    @pl.when(s == _NS - 1)
    def _():
        pltpu.sync_copy(ztab.at[pl.ds(0, _SEG_REM)],
                        table_sp.at[pl.ds(_NS * _SEG_PER_TILE, _SEG_REM)])

    @pl.loop(0, _SEG_PER_TILE // _ZROWS)
    def _(t):
        pltpu.make_async_copy(
            ztab, table_sp.at[pl.ds(s * _SEG_PER_TILE + t * _ZROWS, _ZROWS)],
            sem1).wait()

    iload.wait()
    plsc.subcore_barrier()

    row0 = w * _ROWS_PER_W

    def start(t, buf, sem):
        pltpu.async_copy(vals_hbm.at[pl.ds(row0 + t * _CHUNK, _CHUNK)],
                         buf, sem)

    def wait_load(t, buf, sem):
        pltpu.make_async_copy(vals_hbm.at[pl.ds(row0 + t * _CHUNK, _CHUNK)],
                              buf, sem).wait()

    def hist_update(t):
        @pl.loop(0, _CHUNK // 16)
        def _(j):
            idxv = iball[t, pl.ds(j * 16, 16)]
            plsc.addupdate_scatter(hist, [idxv], ones16)

    start(0, vbuf0, sem0)
    start(1, vbuf1, sem1)

    @pl.loop(0, _NCHUNK - 1, step=2)
    def _(t):
        wait_load(t, vbuf0, sem0)
        sc0 = pltpu.async_copy(vbuf0, table_sp.at[iball.at[t]], sems0,
                               add=True)
        hist_update(t)
        wait_load(t + 1, vbuf1, sem1)
        sc1 = pltpu.async_copy(vbuf1, table_sp.at[iball.at[t + 1]], sems1,
                               add=True)
        hist_update(t + 1)
        sc0.wait()
        start(t + 2, vbuf0, sem0)
        sc1.wait()

        @pl.when(t + 3 < _NCHUNK)
        def _():
            start(t + 3, vbuf1, sem1)

    wait_load(_NCHUNK - 1, vbuf0, sem0)
    pltpu.sync_copy(vbuf0, table_sp.at[iball.at[_NCHUNK - 1]], add=True)
    hist_update(_NCHUNK - 1)

    pltpu.sync_copy(hist, cnt_hbm.at[c, s])
    plsc.subcore_barrier()

    pltpu.sync_copy(table_sp.at[pl.ds(s * _SEG_PER_TILE, _SEG_PER_TILE)],
                    part_hbm.at[c, pl.ds(s * _SEG_PER_TILE, _SEG_PER_TILE)])

    @pl.when(s == _NS - 1)
    def _():
        pltpu.sync_copy(table_sp.at[pl.ds(_NS * _SEG_PER_TILE, _SEG_REM)],
                        part_hbm.at[c, pl.ds(_NS * _SEG_PER_TILE, _SEG_REM)])


def _tc_merge(part_ref, cnt_ref, out_ref):
    p = part_ref[0] + part_ref[1]
    n = jnp.sum(cnt_ref[...], axis=1, keepdims=True) + jnp.float32(_EPS)
    out_ref[...] = p * (1.0 / n)


@jax.jit
def kernel(tens_values, tens_indices):
    seg = tens_indices[1].astype(jnp.int32)
    vals = tens_values.astype(jnp.float32)

    mesh = plsc.VectorSubcoreMesh(core_axis_name="c", subcore_axis_name="s")
    cp = pltpu.CompilerParams()
    if "needs_layout_passes" in pltpu.CompilerParams.__dataclass_fields__:
        cp = dataclasses.replace(cp, needs_layout_passes=False)
    phase1 = pl.kernel(
        _sc_phase1,
        out_type=[
            jax.ShapeDtypeStruct((_NC, _N_SEG, _D), jnp.float32),
            jax.ShapeDtypeStruct((_NC, _NS, _N_SEG), jnp.float32),
        ],
        mesh=mesh,
        compiler_params=cp,
        scratch_types=[
            pltpu.VMEM_SHARED((_N_SEG, _D), jnp.float32),
            pltpu.VMEM((_CHUNK, _D), jnp.float32),
            pltpu.VMEM((_CHUNK, _D), jnp.float32),
            pltpu.VMEM((_NCHUNK, _CHUNK), jnp.int32),
            pltpu.VMEM((_N_SEG,), jnp.float32),
            pltpu.VMEM((_ZROWS, _D), jnp.float32),
            pltpu.SemaphoreType.DMA,
            pltpu.SemaphoreType.DMA,
            pltpu.SemaphoreType.DMA,
            pltpu.SemaphoreType.DMA,
        ],
    )
    part, cnt = phase1(vals, seg.reshape(_NW, _NCHUNK, _CHUNK))

    cnt_t = jnp.transpose(cnt.reshape(_NW, _N_SEG))

    out = pl.pallas_call(
        _tc_merge,
        out_shape=jax.ShapeDtypeStruct((_N_SEG, _D), jnp.float32),
    )(part, cnt_t)
    return out


# speedup vs baseline: 1.2267x; 1.2267x over previous
import dataclasses
import functools

import jax
import jax.numpy as jnp
from jax import lax
from jax.experimental import pallas as pl
from jax.experimental.pallas import tpu as pltpu
from jax.experimental.pallas import tpu_sc as plsc

_N_SEG = 10000
_NNZ = 320000
_D = 128
_EPS = 1e-16

_NC = 2
_NS = 16
_NW = _NC * _NS
_ROWS_PER_W = _NNZ // _NW
_CHUNK = 80
_NCHUNK = _ROWS_PER_W // _CHUNK
_SEG_PER_TILE = 624
_SEG_REM = _N_SEG - _NS * _SEG_PER_TILE
_ZROWS = 16


def _sc_phase1(vals_hbm, seg_hbm, part_hbm, cnt_hbm,
               table_sp, vbuf0, vbuf1, iball, hist, ztab,
               sem0, sem1, sems0, sems1):
    c = lax.axis_index("c")
    s = lax.axis_index("s")
    w = c * _NS + s

    zero16 = jnp.zeros((16,), jnp.float32)
    ones16 = zero16 + 1.0

    iload = pltpu.async_copy(seg_hbm.at[w], iball, sem0)

    @pl.loop(0, _ZROWS)
    def _(i):
        @pl.loop(0, _D // 16)
        def _(j):
            ztab[i, pl.ds(j * 16, 16)] = zero16

    @pl.loop(0, _N_SEG // 16)
    def _(i):
        hist[pl.ds(i * 16, 16)] = zero16

    @pl.loop(0, _SEG_PER_TILE // _ZROWS)
    def _(t):
        pltpu.async_copy(ztab,
                         table_sp.at[pl.ds(s * _SEG_PER_TILE + t * _ZROWS,
                                           _ZROWS)], sem1)

    @pl.when(s == _NS - 1)
    def _():
        pltpu.sync_copy(ztab.at[pl.ds(0, _SEG_REM)],
                        table_sp.at[pl.ds(_NS * _SEG_PER_TILE, _SEG_REM)])

    @pl.loop(0, _SEG_PER_TILE // _ZROWS)
    def _(t):
        pltpu.make_async_copy(
            ztab, table_sp.at[pl.ds(s * _SEG_PER_TILE + t * _ZROWS, _ZROWS)],
            sem1).wait()

    iload.wait()
    plsc.subcore_barrier()

    row0 = w * _ROWS_PER_W

    def start(t, buf, sem):
        pltpu.async_copy(vals_hbm.at[pl.ds(row0 + t * _CHUNK, _CHUNK)],
                         buf, sem)

    def wait_load(t, buf, sem):
        pltpu.make_async_copy(vals_hbm.at[pl.ds(row0 + t * _CHUNK, _CHUNK)],
                              buf, sem).wait()

    def hist_update(t):
        @pl.loop(0, _CHUNK // 16)
        def _(j):
            idxv = iball[t, pl.ds(j * 16, 16)]
            plsc.addupdate_scatter(hist, [idxv], ones16)

    def finish(t, buf, sem):
        wait_load(t, buf, sem)
        pltpu.sync_copy(buf, table_sp.at[iball.at[t]], add=True)
        hist_update(t)

    start(0, vbuf0, sem0)

    @pl.loop(0, _NCHUNK - 1, step=2)
    def _(t):
        start(t + 1, vbuf1, sem1)
        finish(t, vbuf0, sem0)
        start(t + 2, vbuf0, sem0)
        finish(t + 1, vbuf1, sem1)

    finish(_NCHUNK - 1, vbuf0, sem0)

    pltpu.sync_copy(hist, cnt_hbm.at[c, s])
    plsc.subcore_barrier()

    pltpu.sync_copy(table_sp.at[pl.ds(s * _SEG_PER_TILE, _SEG_PER_TILE)],
                    part_hbm.at[c, pl.ds(s * _SEG_PER_TILE, _SEG_PER_TILE)])

    @pl.when(s == _NS - 1)
    def _():
        pltpu.sync_copy(table_sp.at[pl.ds(_NS * _SEG_PER_TILE, _SEG_REM)],
                        part_hbm.at[c, pl.ds(_NS * _SEG_PER_TILE, _SEG_REM)])


def _tc_merge(part_ref, cnt_ref, out_ref):
    p = part_ref[0] + part_ref[1]
    n = jnp.sum(cnt_ref[...], axis=1, keepdims=True) + jnp.float32(_EPS)
    out_ref[...] = p * (1.0 / n)


@jax.jit
def kernel(tens_values, tens_indices):
    seg = tens_indices[1].astype(jnp.int32)
    vals = tens_values.astype(jnp.float32)

    mesh = plsc.VectorSubcoreMesh(core_axis_name="c", subcore_axis_name="s")
    cp = pltpu.CompilerParams()
    if "needs_layout_passes" in pltpu.CompilerParams.__dataclass_fields__:
        cp = dataclasses.replace(cp, needs_layout_passes=False)
    phase1 = pl.kernel(
        _sc_phase1,
        out_type=[
            jax.ShapeDtypeStruct((_NC, _N_SEG, _D), jnp.float32),
            jax.ShapeDtypeStruct((_NC, _NS, _N_SEG), jnp.float32),
        ],
        mesh=mesh,
        compiler_params=cp,
        scratch_types=[
            pltpu.VMEM_SHARED((_N_SEG, _D), jnp.float32),
            pltpu.VMEM((_CHUNK, _D), jnp.float32),
            pltpu.VMEM((_CHUNK, _D), jnp.float32),
            pltpu.VMEM((_NCHUNK, _CHUNK), jnp.int32),
            pltpu.VMEM((_N_SEG,), jnp.float32),
            pltpu.VMEM((_ZROWS, _D), jnp.float32),
            pltpu.SemaphoreType.DMA,
            pltpu.SemaphoreType.DMA,
            pltpu.SemaphoreType.DMA,
            pltpu.SemaphoreType.DMA,
        ],
    )
    part, cnt = phase1(vals, seg.reshape(_NW, _NCHUNK, _CHUNK))

    cnt_t = jnp.transpose(cnt.reshape(_NW, _N_SEG))

    out = pl.pallas_call(
        _tc_merge,
        out_shape=jax.ShapeDtypeStruct((_N_SEG, _D), jnp.float32),
    )(part, cnt_t)
    return out


# speedup vs baseline: 1.3576x; 1.1067x over previous
import dataclasses
import functools

import jax
import jax.numpy as jnp
from jax import lax
from jax.experimental import pallas as pl
from jax.experimental.pallas import tpu as pltpu
from jax.experimental.pallas import tpu_sc as plsc

_N_SEG = 10000
_NNZ = 320000
_D = 128
_EPS = 1e-16

_NC = 2
_NS = 16
_NW = _NC * _NS
_ROWS_PER_W = _NNZ // _NW
_CHUNK = 80
_NCHUNK = _ROWS_PER_W // _CHUNK
_SEG_PER_TILE = 624
_SEG_REM = _N_SEG - _NS * _SEG_PER_TILE
_ZROWS = 16


def _sc_phase1(vals_hbm, seg_hbm, part_hbm, cnt_hbm,
               table_sp, vbuf0, vbuf1, iball, hist, ztab,
               sem0, sem1, sems0, sems1):
    c = lax.axis_index("c")
    s = lax.axis_index("s")
    w = c * _NS + s

    zero16 = jnp.zeros((16,), jnp.float32)
    ones16 = zero16 + 1.0

    iload = pltpu.async_copy(seg_hbm.at[1, w], iball, sem0)

    @pl.loop(0, _ZROWS)
    def _(i):
        @pl.loop(0, _D // 16)
        def _(j):
            ztab[i, pl.ds(j * 16, 16)] = zero16

    @pl.loop(0, _N_SEG // 16)
    def _(i):
        hist[pl.ds(i * 16, 16)] = zero16

    @pl.loop(0, _SEG_PER_TILE // _ZROWS)
    def _(t):
        pltpu.async_copy(ztab,
                         table_sp.at[pl.ds(s * _SEG_PER_TILE + t * _ZROWS,
                                           _ZROWS)], sem1)

    @pl.when(s == _NS - 1)
    def _():
        pltpu.sync_copy(ztab.at[pl.ds(0, _SEG_REM)],
                        table_sp.at[pl.ds(_NS * _SEG_PER_TILE, _SEG_REM)])

    @pl.loop(0, _SEG_PER_TILE // _ZROWS)
    def _(t):
        pltpu.make_async_copy(
            ztab, table_sp.at[pl.ds(s * _SEG_PER_TILE + t * _ZROWS, _ZROWS)],
            sem1).wait()

    iload.wait()
    plsc.subcore_barrier()

    row0 = w * _ROWS_PER_W

    def start(t, buf, sem):
        pltpu.async_copy(vals_hbm.at[pl.ds(row0 + t * _CHUNK, _CHUNK)],
                         buf, sem)

    def wait_load(t, buf, sem):
        pltpu.make_async_copy(vals_hbm.at[pl.ds(row0 + t * _CHUNK, _CHUNK)],
                              buf, sem).wait()

    def hist_update(t):
        @pl.loop(0, _CHUNK // 16)
        def _(j):
            idxv = iball[t, pl.ds(j * 16, 16)]
            plsc.addupdate_scatter(hist, [idxv], ones16)

    def finish(t, buf, sem):
        wait_load(t, buf, sem)
        pltpu.sync_copy(buf, table_sp.at[iball.at[t]], add=True)
        hist_update(t)

    start(0, vbuf0, sem0)

    @pl.loop(0, _NCHUNK - 1, step=2)
    def _(t):
        start(t + 1, vbuf1, sem1)
        finish(t, vbuf0, sem0)
        start(t + 2, vbuf0, sem0)
        finish(t + 1, vbuf1, sem1)

    finish(_NCHUNK - 1, vbuf0, sem0)

    pltpu.sync_copy(hist, cnt_hbm.at[c, s])
    plsc.subcore_barrier()

    pltpu.sync_copy(table_sp.at[pl.ds(s * _SEG_PER_TILE, _SEG_PER_TILE)],
                    part_hbm.at[c, pl.ds(s * _SEG_PER_TILE, _SEG_PER_TILE)])

    @pl.when(s == _NS - 1)
    def _():
        pltpu.sync_copy(table_sp.at[pl.ds(_NS * _SEG_PER_TILE, _SEG_REM)],
                        part_hbm.at[c, pl.ds(_NS * _SEG_PER_TILE, _SEG_REM)])


def _tc_merge(part_ref, cnt_ref, out_ref):
    p = part_ref[0] + part_ref[1]
    ones_b = jnp.ones((_NW, _D), jnp.float32)
    n = jax.lax.dot_general(cnt_ref[...], ones_b, (((0,), (0,)), ((), ())),
                            preferred_element_type=jnp.float32)
    out_ref[...] = p / (n + jnp.float32(_EPS))


@jax.jit
def kernel(tens_values, tens_indices):
    seg = tens_indices.astype(jnp.int32).reshape(2, _NW, _NCHUNK, _CHUNK)
    vals = tens_values.astype(jnp.float32)

    mesh = plsc.VectorSubcoreMesh(core_axis_name="c", subcore_axis_name="s")
    cp = pltpu.CompilerParams()
    if "needs_layout_passes" in pltpu.CompilerParams.__dataclass_fields__:
        cp = dataclasses.replace(cp, needs_layout_passes=False)
    phase1 = pl.kernel(
        _sc_phase1,
        out_type=[
            jax.ShapeDtypeStruct((_NC, _N_SEG, _D), jnp.float32),
            jax.ShapeDtypeStruct((_NC, _NS, _N_SEG), jnp.float32),
        ],
        mesh=mesh,
        compiler_params=cp,
        scratch_types=[
            pltpu.VMEM_SHARED((_N_SEG, _D), jnp.float32),
            pltpu.VMEM((_CHUNK, _D), jnp.float32),
            pltpu.VMEM((_CHUNK, _D), jnp.float32),
            pltpu.VMEM((_NCHUNK, _CHUNK), jnp.int32),
            pltpu.VMEM((_N_SEG,), jnp.float32),
            pltpu.VMEM((_ZROWS, _D), jnp.float32),
            pltpu.SemaphoreType.DMA,
            pltpu.SemaphoreType.DMA,
            pltpu.SemaphoreType.DMA,
            pltpu.SemaphoreType.DMA,
        ],
    )
    part, cnt = phase1(vals, seg)

    out = pl.pallas_call(
        _tc_merge,
        out_shape=jax.ShapeDtypeStruct((_N_SEG, _D), jnp.float32),
    )(part, cnt.reshape(_NW, _N_SEG))
    return out


# speedup vs baseline: 1.3713x; 1.0101x over previous
import dataclasses
import functools

import jax
import jax.numpy as jnp
from jax import lax
from jax.experimental import pallas as pl
from jax.experimental.pallas import tpu as pltpu
from jax.experimental.pallas import tpu_sc as plsc

_N_SEG = 10000
_NNZ = 320000
_D = 128
_EPS = 1e-16

_NC = 2
_NS = 16
_NW = _NC * _NS
_ROWS_PER_W = _NNZ // _NW
_CHUNK = 80
_NCHUNK = _ROWS_PER_W // _CHUNK
_SEG_PER_TILE = 624
_SEG_REM = _N_SEG - _NS * _SEG_PER_TILE
_ZROWS = 16


def _sc_phase1(vals_hbm, seg_hbm, part_hbm, cnt_hbm,
               table_sp, vbuf0, vbuf1, iball, hist, ztab, sem0, sem1, sems0):
    c = lax.axis_index("c")
    s = lax.axis_index("s")
    w = c * _NS + s

    zero16 = jnp.zeros((16,), jnp.float32)
    ones16 = zero16 + 1.0

    iload = pltpu.async_copy(seg_hbm.at[1, w], iball, sem0)

    @pl.loop(0, _ZROWS)
    def _(i):
        @pl.loop(0, _D // 16)
        def _(j):
            ztab[i, pl.ds(j * 16, 16)] = zero16

    @pl.loop(0, _N_SEG // 16)
    def _(i):
        hist[pl.ds(i * 16, 16)] = zero16

    @pl.loop(0, _SEG_PER_TILE // _ZROWS)
    def _(t):
        pltpu.async_copy(ztab,
                         table_sp.at[pl.ds(s * _SEG_PER_TILE + t * _ZROWS,
                                           _ZROWS)], sem1)

    @pl.when(s == _NS - 1)
    def _():
        pltpu.sync_copy(ztab.at[pl.ds(0, _SEG_REM)],
                        table_sp.at[pl.ds(_NS * _SEG_PER_TILE, _SEG_REM)])

    @pl.loop(0, _SEG_PER_TILE // _ZROWS)
    def _(t):
        pltpu.make_async_copy(
            ztab, table_sp.at[pl.ds(s * _SEG_PER_TILE + t * _ZROWS, _ZROWS)],
            sem1).wait()

    iload.wait()
    plsc.subcore_barrier()

    row0 = w * _ROWS_PER_W

    def start(t, buf, sem):
        pltpu.async_copy(vals_hbm.at[pl.ds(row0 + t * _CHUNK, _CHUNK)],
                         buf, sem)

    def wait_load(t, buf, sem):
        pltpu.make_async_copy(vals_hbm.at[pl.ds(row0 + t * _CHUNK, _CHUNK)],
                              buf, sem).wait()

    def hist_update(t):
        @pl.loop(0, _CHUNK // 16)
        def _(j):
            idxv = iball[t, pl.ds(j * 16, 16)]
            plsc.addupdate_scatter(hist, [idxv], ones16)

    def finish(t, buf, sem):
        wait_load(t, buf, sem)
        sc = pltpu.async_copy(buf, table_sp.at[iball.at[t]], sems0, add=True)
        hist_update(t)
        sc.wait()

    start(0, vbuf0, sem0)

    @pl.loop(0, _NCHUNK - 1, step=2)
    def _(t):
        start(t + 1, vbuf1, sem1)
        finish(t, vbuf0, sem0)
        start(t + 2, vbuf0, sem0)
        finish(t + 1, vbuf1, sem1)

    finish(_NCHUNK - 1, vbuf0, sem0)

    pltpu.sync_copy(hist, cnt_hbm.at[c, s])
    plsc.subcore_barrier()

    pltpu.sync_copy(table_sp.at[pl.ds(s * _SEG_PER_TILE, _SEG_PER_TILE)],
                    part_hbm.at[c, pl.ds(s * _SEG_PER_TILE, _SEG_PER_TILE)])

    @pl.when(s == _NS - 1)
    def _():
        pltpu.sync_copy(table_sp.at[pl.ds(_NS * _SEG_PER_TILE, _SEG_REM)],
                        part_hbm.at[c, pl.ds(_NS * _SEG_PER_TILE, _SEG_REM)])


def _tc_merge(part_ref, cnt_ref, out_ref):
    p = part_ref[0] + part_ref[1]
    ones_b = jnp.ones((_NW, _D), jnp.float32)
    n = jax.lax.dot_general(cnt_ref[...], ones_b, (((0,), (0,)), ((), ())),
                            preferred_element_type=jnp.float32)
    out_ref[...] = p / (n + jnp.float32(_EPS))


@jax.jit
def kernel(tens_values, tens_indices):
    seg = tens_indices.astype(jnp.int32).reshape(2, _NW, _NCHUNK, _CHUNK)
    vals = tens_values.astype(jnp.float32)

    mesh = plsc.VectorSubcoreMesh(core_axis_name="c", subcore_axis_name="s")
    cp = pltpu.CompilerParams()
    if "needs_layout_passes" in pltpu.CompilerParams.__dataclass_fields__:
        cp = dataclasses.replace(cp, needs_layout_passes=False)
    phase1 = pl.kernel(
        _sc_phase1,
        out_type=[
            jax.ShapeDtypeStruct((_NC, _N_SEG, _D), jnp.float32),
            jax.ShapeDtypeStruct((_NC, _NS, _N_SEG), jnp.float32),
        ],
        mesh=mesh,
        compiler_params=cp,
        scratch_types=[
            pltpu.VMEM_SHARED((_N_SEG, _D), jnp.float32),
            pltpu.VMEM((_CHUNK, _D), jnp.float32),
            pltpu.VMEM((_CHUNK, _D), jnp.float32),
            pltpu.VMEM((_NCHUNK, _CHUNK), jnp.int32),
            pltpu.VMEM((_N_SEG,), jnp.float32),
            pltpu.VMEM((_ZROWS, _D), jnp.float32),
            pltpu.SemaphoreType.DMA,
            pltpu.SemaphoreType.DMA,
            pltpu.SemaphoreType.DMA,
        ],
    )
    part, cnt = phase1(vals, seg)

    out = pl.pallas_call(
        _tc_merge,
        out_shape=jax.ShapeDtypeStruct((_N_SEG, _D), jnp.float32),
    )(part, cnt.reshape(_NW, _N_SEG))
    return out
